# Initial kernel scaffold; baseline (speedup 1.0000x reference)
import jax
import jax.numpy as jnp
from jax.experimental import pallas as pl


def kernel(x, hyperedge_index, W, b, hyperedge_weight):
    raise NotImplementedError("write your pallas kernel here")


# speedup vs baseline: 4.3358x; 4.3358x over previous
import functools

import jax
import jax.numpy as jnp
from jax import lax
from jax.experimental import pallas as pl
from jax.experimental.pallas import tpu as pltpu
from jax.experimental.pallas import tpu_sc as plsc

N_NODES = 10000
N_HE = 10000
D_IN = 128
DH = 64
R = 10240
PAD_BIN = 10000
INC = 320000
INC_PAD = 327680
IDX_ROWS = 2560
NS = 16
RT = R // NS
IRT = IDX_ROWS // NS
STAGE = 16
NSTAGE = IRT // STAGE
EPS = 1e-6


def _mm_body(x_ref, w_ref, b_ref, o_ref):
    o_ref[0] = lax.dot_general(
        x_ref[...], w_ref[...], (((1,), (1,)), ((), ())),
        preferred_element_type=jnp.float32) + b_ref[0, 0][None, :]


def _transform(x_pad, W, b2):
    return pl.pallas_call(
        _mm_body,
        grid=(2, 4),
        in_specs=[
            pl.BlockSpec((R // 4, 128), lambda c, r: (r, 0)),
            pl.BlockSpec((DH, 128), lambda c, r: (c, 0)),
            pl.BlockSpec((1, 1, DH), lambda c, r: (c, 0, 0)),
        ],
        out_specs=pl.BlockSpec((1, R // 4, DH), lambda c, r: (c, r, 0)),
        out_shape=jax.ShapeDtypeStruct((2, R, DH), jnp.float32),
    )(x_pad, W, b2)


def _sc_body(xt_ref, idxn_ref, idxe_ref, w_ref,
             he_ref, out_ref,
             he_sh, out_sh, b_sh, d_sh,
             idxn_v, idxe_v, buf0, buf1, ones_v, z_v, chunk_v, svec, wvec,
             gsem0, gsem1):
    cid = lax.axis_index("c")
    sid = lax.axis_index("s")
    row0 = sid * RT
    ib0 = sid * IRT

    zeros16 = jnp.zeros((16,), jnp.float32)
    ones16 = jnp.ones((16,), jnp.float32)
    for k in range(8):
        ones_v[pl.ds(16 * k, 16)] = ones16

    def zrow(i, c):
        for k in range(DH // 16):
            z_v[i, pl.ds(16 * k, 16)] = zeros16
        return c
    lax.fori_loop(0, 64, zrow, 0)

    def zacc(j, c):
        r = row0 + j * 64
        pltpu.sync_copy(z_v, he_sh.at[pl.ds(r, 64)])
        pltpu.sync_copy(z_v, out_sh.at[pl.ds(r, 64)])
        pltpu.sync_copy(z_v.at[0], b_sh.at[pl.ds(r, 64)])
        pltpu.sync_copy(z_v.at[0], d_sh.at[pl.ds(r, 64)])
        return c
    lax.fori_loop(0, RT // 64, zacc, 0)
    plsc.subcore_barrier()

    def p1(jo, c):
        rbase = ib0 + jo * STAGE
        pltpu.sync_copy(idxn_ref.at[pl.ds(rbase, STAGE)], idxn_v)
        pltpu.sync_copy(idxe_ref.at[pl.ds(rbase, STAGE)], idxe_v)
        pend = pltpu.async_copy(xt_ref.at[cid].at[idxn_v.at[0]], buf0, gsem0)
        for j in range(STAGE):
            cur = pend
            curbuf = buf0 if j % 2 == 0 else buf1
            if j + 1 < STAGE:
                pend = pltpu.async_copy(
                    xt_ref.at[cid].at[idxn_v.at[j + 1]],
                    buf1 if j % 2 == 0 else buf0,
                    gsem1 if j % 2 == 0 else gsem0)
            pltpu.sync_copy(ones_v, d_sh.at[idxn_v.at[j]], add=True)
            pltpu.sync_copy(ones_v, b_sh.at[idxe_v.at[j]], add=True)
            cur.wait()
            pltpu.sync_copy(curbuf, he_sh.at[idxe_v.at[j]], add=True)
        return c
    lax.fori_loop(0, NSTAGE, p1, 0)
    plsc.subcore_barrier()

    def scale_he(j, c):
        r = row0 + j * 64
        pltpu.sync_copy(he_sh.at[pl.ds(r, 64)], chunk_v)
        pltpu.sync_copy(b_sh.at[pl.ds(r, 64)], svec)
        pltpu.sync_copy(w_ref.at[pl.ds(r, 64)], wvec)
        for k in range(4):
            sl = pl.ds(16 * k, 16)
            svec[sl] = wvec[sl] / (svec[sl] + EPS)

        def grpmul(g, c2):
            s16 = svec[pl.ds(16 * g, 16)]
            for rr in range(16):
                srow = jnp.broadcast_to(s16[rr], (16,))
                row = 16 * g + rr
                for k in range(DH // 16):
                    sl = pl.ds(16 * k, 16)
                    chunk_v[row, sl] = chunk_v[row, sl] * srow
            return c2
        lax.fori_loop(0, 4, grpmul, 0)
        pltpu.sync_copy(chunk_v, he_ref.at[cid].at[pl.ds(r, 64)])
        return c
    lax.fori_loop(0, RT // 64, scale_he, 0)
    plsc.subcore_barrier()

    def p2(jo, c):
        rbase = ib0 + jo * STAGE
        pltpu.sync_copy(idxn_ref.at[pl.ds(rbase, STAGE)], idxn_v)
        pltpu.sync_copy(idxe_ref.at[pl.ds(rbase, STAGE)], idxe_v)
        pend = pltpu.async_copy(he_ref.at[cid].at[idxe_v.at[0]], buf0, gsem0)
        for j in range(STAGE):
            cur = pend
            curbuf = buf0 if j % 2 == 0 else buf1
            if j + 1 < STAGE:
                pend = pltpu.async_copy(
                    he_ref.at[cid].at[idxe_v.at[j + 1]],
                    buf1 if j % 2 == 0 else buf0,
                    gsem1 if j % 2 == 0 else gsem0)
            cur.wait()
            pltpu.sync_copy(curbuf, out_sh.at[idxn_v.at[j]], add=True)
        return c
    lax.fori_loop(0, NSTAGE, p2, 0)
    plsc.subcore_barrier()

    def scale_out(j, c):
        r = row0 + j * 64
        pltpu.sync_copy(out_sh.at[pl.ds(r, 64)], chunk_v)
        pltpu.sync_copy(d_sh.at[pl.ds(r, 64)], svec)
        for k in range(4):
            sl = pl.ds(16 * k, 16)
            svec[sl] = 1.0 / (svec[sl] + EPS)

        def grpmul(g, c2):
            s16 = svec[pl.ds(16 * g, 16)]
            for rr in range(16):
                srow = jnp.broadcast_to(s16[rr], (16,))
                row = 16 * g + rr
                for k in range(DH // 16):
                    sl = pl.ds(16 * k, 16)
                    chunk_v[row, sl] = chunk_v[row, sl] * srow
            return c2
        lax.fori_loop(0, 4, grpmul, 0)
        pltpu.sync_copy(chunk_v, out_ref.at[cid].at[pl.ds(r, 64)])
        return c
    lax.fori_loop(0, RT // 64, scale_out, 0)


_sc_call = functools.partial(
    pl.kernel,
    out_type=(
        jax.ShapeDtypeStruct((2, R, DH), jnp.float32),
        jax.ShapeDtypeStruct((2, R, DH), jnp.float32),
    ),
    mesh=plsc.VectorSubcoreMesh(core_axis_name="c", subcore_axis_name="s"),
    compiler_params=pltpu.CompilerParams(use_tc_tiling_on_sc=False),
    scratch_types=[
        pltpu.VMEM_SHARED((R, DH), jnp.float32),
        pltpu.VMEM_SHARED((R, DH), jnp.float32),
        pltpu.VMEM_SHARED((R,), jnp.float32),
        pltpu.VMEM_SHARED((R,), jnp.float32),
        pltpu.VMEM((STAGE, 128), jnp.int32),
        pltpu.VMEM((STAGE, 128), jnp.int32),
        pltpu.VMEM((128, DH), jnp.float32),
        pltpu.VMEM((128, DH), jnp.float32),
        pltpu.VMEM((128,), jnp.float32),
        pltpu.VMEM((64, DH), jnp.float32),
        pltpu.VMEM((64, DH), jnp.float32),
        pltpu.VMEM((64,), jnp.float32),
        pltpu.VMEM((64,), jnp.float32),
        pltpu.SemaphoreType.DMA,
        pltpu.SemaphoreType.DMA,
    ],
)(_sc_body)


def kernel(x, hyperedge_index, W, b, hyperedge_weight):
    x_pad = jnp.pad(x, ((0, R - N_NODES), (0, 0)))
    pad = jnp.full((INC_PAD - INC,), PAD_BIN, jnp.int32)
    idxn = jnp.concatenate([hyperedge_index[0], pad]).reshape(IDX_ROWS, 128)
    idxe = jnp.concatenate([hyperedge_index[1], pad]).reshape(IDX_ROWS, 128)
    w_pad = jnp.pad(hyperedge_weight, (0, R - N_HE))
    b2 = b.reshape(2, 1, DH)
    xt = _transform(x_pad, W, b2)
    _, out2 = _sc_call(xt, idxn, idxe, w_pad)
    return jnp.concatenate([out2[0, :N_NODES], out2[1, :N_NODES]], axis=1)
